# baseline (speedup 1.0000x reference)
import functools

import jax
import jax.numpy as jnp
from jax import lax
from jax.experimental import pallas as pl
from jax.experimental.pallas import tpu as pltpu
from jax.experimental.pallas import tpu_sc as plsc

D = 1024
M = 8192
S = 2048
MAX_K = 8
MIN_K = 2
TAU = 0.9

T_TILE = 256
KEY_TILE = 1024
V_TILE = 640
NEG = -3.0e38
HI = jax.lax.Precision.HIGHEST

NW = 32
ROWS_PER_W = S // NW


def _sc_embed_body(table_hbm, idx_hbm, out_hbm, idx_v, rows_v, sem):
    wid = lax.axis_index("s") * 2 + lax.axis_index("c")
    base = wid * ROWS_PER_W
    pltpu.sync_copy(idx_hbm.at[pl.ds(base, ROWS_PER_W)], idx_v)
    pltpu.async_copy(table_hbm.at[idx_v], rows_v, sem).wait()
    pltpu.sync_copy(rows_v, out_hbm.at[pl.ds(base, ROWS_PER_W)])


def _sc_embed_gather(table, idx):
    mesh = plsc.VectorSubcoreMesh(core_axis_name="c", subcore_axis_name="s")
    f = pl.kernel(
        _sc_embed_body,
        mesh=mesh,
        out_type=jax.ShapeDtypeStruct((S, D), jnp.float32),
        scratch_types=[
            pltpu.VMEM((ROWS_PER_W,), jnp.int32),
            pltpu.VMEM((ROWS_PER_W, D), jnp.float32),
            pltpu.SemaphoreType.DMA,
        ],
    )
    return f(table, idx)


def _sc_vals_body(table_hbm, idx_hbm, out_hbm, idx_v, rows_v, sem):
    wid = lax.axis_index("s") * 2 + lax.axis_index("c")
    base = wid * ROWS_PER_W * MAX_K
    pltpu.sync_copy(idx_hbm.at[pl.ds(base, ROWS_PER_W * MAX_K)], idx_v)
    for c in range(MAX_K):
        pltpu.async_copy(table_hbm.at[idx_v.at[pl.ds(c * 64, 64)]],
                         rows_v, sem).wait()
        pltpu.sync_copy(rows_v, out_hbm.at[pl.ds(base + c * 64, 64)])


def _sc_vals_gather(table, idx_flat):
    mesh = plsc.VectorSubcoreMesh(core_axis_name="c", subcore_axis_name="s")
    f = pl.kernel(
        _sc_vals_body,
        mesh=mesh,
        out_type=jax.ShapeDtypeStruct((S * MAX_K, D), jnp.float32),
        scratch_types=[
            pltpu.VMEM((ROWS_PER_W * MAX_K,), jnp.int32),
            pltpu.VMEM((64, D), jnp.float32),
            pltpu.SemaphoreType.DMA,
        ],
    )
    return f(table, idx_flat)


def _router_body(h_ref, scale_ref, bias_ref, wq_ref, bq_ref, keys_ref,
                 gated_ref, idx_ref, q_scr, rv_scr, ri_scr):
    j = pl.program_id(1)
    nj = pl.num_programs(1)

    @pl.when(j == 0)
    def _ln_q():
        h = h_ref[...]
        mu = jnp.mean(h, axis=-1, keepdims=True)
        var = jnp.mean((h - mu) ** 2, axis=-1, keepdims=True)
        hn = (h - mu) / jnp.sqrt(var + 1e-6) * scale_ref[...] + bias_ref[...]
        q_scr[...] = jnp.dot(hn, wq_ref[...],
                             preferred_element_type=jnp.float32) + bq_ref[...]
        rv_scr[...] = jnp.full_like(rv_scr, NEG)
        ri_scr[...] = jnp.zeros_like(ri_scr)

    s = jax.lax.dot_general(q_scr[...], keys_ref[...],
                            (((1,), (1,)), ((), ())),
                            preferred_element_type=jnp.float32) * (1.0 / 32.0)

    iota = jax.lax.broadcasted_iota(jnp.int32, s.shape, 1)
    bv, bi = [], []
    for _ in range(MAX_K):
        m = jnp.max(s, axis=1, keepdims=True)
        am = jnp.min(jnp.where(s == m, iota, jnp.int32(1 << 30)),
                     axis=1, keepdims=True)
        bv.append(m)
        bi.append(am + j * KEY_TILE)
        s = jnp.where(iota == am, NEG, s)

    cv = jnp.concatenate([rv_scr[...]] + bv, axis=1)
    ci = jnp.concatenate([ri_scr[...]] + bi, axis=1)
    pos = jax.lax.broadcasted_iota(jnp.int32, cv.shape, 1)
    nv, ni = [], []
    for _ in range(MAX_K):
        m = jnp.max(cv, axis=1, keepdims=True)
        ap = jnp.min(jnp.where(cv == m, pos, jnp.int32(1 << 30)),
                     axis=1, keepdims=True)
        nv.append(m)
        ni.append(jnp.sum(jnp.where(pos == ap, ci, 0), axis=1, keepdims=True))
        cv = jnp.where(pos == ap, NEG, cv)
    rv_scr[...] = jnp.concatenate(nv, axis=1)
    ri_scr[...] = jnp.concatenate(ni, axis=1)

    @pl.when(j == nj - 1)
    def _gate():
        tv = rv_scr[...]
        e = jnp.exp(tv - tv[:, 0:1])
        p = e / jnp.sum(e, axis=1, keepdims=True)
        cums = [p[:, 0:1]]
        for k in range(1, MAX_K - 1):
            cums.append(cums[-1] + p[:, k:k + 1])
        prev = jnp.concatenate([jnp.zeros_like(cums[0])] + cums, axis=1)
        lane = jax.lax.broadcasted_iota(jnp.int32, p.shape, 1)
        keep = (lane < MIN_K) | (prev < TAU)
        g = p * keep.astype(jnp.float32)
        g = g / (jnp.sum(g, axis=1, keepdims=True) + 1e-9)
        gated_ref[...] = g
        idx_ref[...] = ri_scr[...]


def _router(h, lp):
    grid = (S // T_TILE, M // KEY_TILE)
    return pl.pallas_call(
        _router_body,
        grid=grid,
        in_specs=[
            pl.BlockSpec((T_TILE, D), lambda t, j: (t, 0)),
            pl.BlockSpec((1, D), lambda t, j: (0, 0)),
            pl.BlockSpec((1, D), lambda t, j: (0, 0)),
            pl.BlockSpec((D, D), lambda t, j: (0, 0)),
            pl.BlockSpec((1, D), lambda t, j: (0, 0)),
            pl.BlockSpec((KEY_TILE, D), lambda t, j: (j, 0)),
        ],
        out_specs=[
            pl.BlockSpec((T_TILE, MAX_K), lambda t, j: (t, 0)),
            pl.BlockSpec((T_TILE, MAX_K), lambda t, j: (t, 0)),
        ],
        out_shape=[
            jax.ShapeDtypeStruct((S, MAX_K), jnp.float32),
            jax.ShapeDtypeStruct((S, MAX_K), jnp.int32),
        ],
        scratch_shapes=[
            pltpu.VMEM((T_TILE, D), jnp.float32),
            pltpu.VMEM((T_TILE, MAX_K), jnp.float32),
            pltpu.VMEM((T_TILE, MAX_K), jnp.int32),
        ],
        compiler_params=pltpu.CompilerParams(
            dimension_semantics=("parallel", "arbitrary")),
    )(h, lp['ln_scale'].reshape(1, D), lp['ln_bias'].reshape(1, D),
      lp['Wq'], lp['bq'].reshape(1, D), lp['keys'])


def _combine_body(vals_ref, g_ref, h_ref, wo_ref, bo_ref, out_ref):
    g = g_ref[...].astype(jnp.bfloat16).astype(jnp.float32)
    def vk(k):
        return vals_ref[:, k, :].astype(jnp.bfloat16).astype(jnp.float32)
    mem = vk(0) * g[:, 0:1]
    for k in range(1, MAX_K):
        mem = mem + vk(k) * g[:, k:k + 1]
    out_ref[...] = (jnp.dot(mem, wo_ref[...],
                            preferred_element_type=jnp.float32)
                    + bo_ref[...] + h_ref[...])


def _combine(vals, gated, h, lp):
    grid = (S // T_TILE,)
    return pl.pallas_call(
        _combine_body,
        grid=grid,
        in_specs=[
            pl.BlockSpec((T_TILE, MAX_K, D), lambda t: (t, 0, 0)),
            pl.BlockSpec((T_TILE, MAX_K), lambda t: (t, 0)),
            pl.BlockSpec((T_TILE, D), lambda t: (t, 0)),
            pl.BlockSpec((D, D), lambda t: (0, 0)),
            pl.BlockSpec((1, D), lambda t: (0, 0)),
        ],
        out_specs=pl.BlockSpec((T_TILE, D), lambda t: (t, 0)),
        out_shape=jax.ShapeDtypeStruct((S, D), jnp.float32),
        compiler_params=pltpu.CompilerParams(
            dimension_semantics=("parallel",)),
    )(vals, gated, h, lp['Wo'], lp['bo'].reshape(1, D))


def _final_body(h_ref, scale_ref, bias_ref, wout_ref, bout_ref, out_ref, hn_scr):
    v = pl.program_id(0)
    t = pl.program_id(1)
    rows = pl.ds(t * T_TILE, T_TILE)

    @pl.when(v == 0)
    def _ln():
        h = h_ref[...]
        mu = jnp.mean(h, axis=-1, keepdims=True)
        var = jnp.mean((h - mu) ** 2, axis=-1, keepdims=True)
        hn_scr[rows, :] = ((h - mu) / jnp.sqrt(var + 1e-6) * scale_ref[...]
                           + bias_ref[...])

    out_ref[...] = (jnp.dot(hn_scr[rows, :], wout_ref[...],
                            preferred_element_type=jnp.float32) + bout_ref[...])


def _final(h, params):
    vocab = params['Wout'].shape[1]
    grid = (vocab // V_TILE, S // T_TILE)
    return pl.pallas_call(
        _final_body,
        grid=grid,
        in_specs=[
            pl.BlockSpec((T_TILE, D), lambda v, t: (t, 0)),
            pl.BlockSpec((1, D), lambda v, t: (0, 0)),
            pl.BlockSpec((1, D), lambda v, t: (0, 0)),
            pl.BlockSpec((D, V_TILE), lambda v, t: (0, v)),
            pl.BlockSpec((1, V_TILE), lambda v, t: (0, v)),
        ],
        out_specs=pl.BlockSpec((T_TILE, V_TILE), lambda v, t: (t, v)),
        out_shape=jax.ShapeDtypeStruct((S, vocab), jnp.float32),
        scratch_shapes=[pltpu.VMEM((S, D), jnp.float32)],
        compiler_params=pltpu.CompilerParams(
            dimension_semantics=("arbitrary", "arbitrary")),
    )(h, params['f_ln_scale'].reshape(1, D), params['f_ln_bias'].reshape(1, D),
      params['Wout'], params['bout'].reshape(1, vocab))


def _aux_body(idx_ref, g_ref, aux_ref, imp_scr):
    t = pl.program_id(0)
    nt = pl.num_programs(0)

    @pl.when(t == 0)
    def _init():
        imp_scr[...] = jnp.zeros_like(imp_scr)

    idx = idx_ref[...]
    g = g_ref[...]
    hi = jax.lax.shift_right_logical(idx, 7)
    lo = jax.lax.bitwise_and(idx, 127)
    acols, bcols = [], []
    for k in range(MAX_K):
        hi_k, lo_k, g_k = hi[:, k:k + 1], lo[:, k:k + 1], g[:, k:k + 1]
        i64 = jax.lax.broadcasted_iota(jnp.int32, (hi.shape[0], 64), 1)
        i128 = jax.lax.broadcasted_iota(jnp.int32, (hi.shape[0], 128), 1)
        acols.append((hi_k == i64).astype(jnp.float32) * g_k)
        bcols.append((lo_k == i128).astype(jnp.float32))
    a = jnp.concatenate(acols, axis=0)
    b = jnp.concatenate(bcols, axis=0)
    imp_scr[...] += jax.lax.dot_general(
        a, b, (((0,), (0,)), ((), ())),
        preferred_element_type=jnp.float32, precision=HI)

    @pl.when(t == nt - 1)
    def _fin():
        imp = imp_scr[...] / float(S)
        mean = jnp.mean(imp)
        aux_ref[...] = (jnp.sum((imp - mean) ** 2) * float(M)).reshape(1, 1)


def _aux(top_idx, gated):
    return pl.pallas_call(
        _aux_body,
        grid=(S // T_TILE,),
        in_specs=[
            pl.BlockSpec((T_TILE, MAX_K), lambda t: (t, 0)),
            pl.BlockSpec((T_TILE, MAX_K), lambda t: (t, 0)),
        ],
        out_specs=pl.BlockSpec((1, 1), lambda t: (0, 0)),
        out_shape=jax.ShapeDtypeStruct((1, 1), jnp.float32),
        scratch_shapes=[pltpu.VMEM((64, 128), jnp.float32)],
        compiler_params=pltpu.CompilerParams(
            dimension_semantics=("arbitrary",)),
    )(top_idx, gated)


def kernel(x, params):
    tokens = x.reshape(-1)
    h = _sc_embed_gather(params['embed'], tokens)
    aux_total = jnp.float32(0.0)
    for lp in params['layers']:
        gated, top_idx = _router(h, lp)
        vals = _sc_vals_gather(lp['values'], top_idx.reshape(-1))
        h = _combine(vals.reshape(S, MAX_K, D), gated, h, lp)
        aux_total = aux_total + _aux(top_idx, gated)[0, 0]
    logits = _final(h, params)
    return logits.reshape(1, S, -1), aux_total


# speedup vs baseline: 1.2431x; 1.2431x over previous
import functools

import jax
import jax.numpy as jnp
from jax import lax
from jax.experimental import pallas as pl
from jax.experimental.pallas import tpu as pltpu
from jax.experimental.pallas import tpu_sc as plsc

D = 1024
M = 8192
S = 2048
MAX_K = 8
MIN_K = 2
TAU = 0.9

T_TILE = 256
KEY_TILE = 1024
V_TILE = 640
NEG = -3.0e38
HI = jax.lax.Precision.HIGHEST

NW = 32
ROWS_PER_W = S // NW


def _sc_embed_body(table_hbm, idx_hbm, out_hbm, idx_v, rows_v, sem):
    wid = lax.axis_index("s") * 2 + lax.axis_index("c")
    base = wid * ROWS_PER_W
    pltpu.sync_copy(idx_hbm.at[pl.ds(base, ROWS_PER_W)], idx_v)
    pltpu.async_copy(table_hbm.at[idx_v], rows_v, sem).wait()
    pltpu.sync_copy(rows_v, out_hbm.at[pl.ds(base, ROWS_PER_W)])


def _sc_embed_gather(table, idx):
    mesh = plsc.VectorSubcoreMesh(core_axis_name="c", subcore_axis_name="s")
    f = pl.kernel(
        _sc_embed_body,
        mesh=mesh,
        out_type=jax.ShapeDtypeStruct((S, D), jnp.float32),
        scratch_types=[
            pltpu.VMEM((ROWS_PER_W,), jnp.int32),
            pltpu.VMEM((ROWS_PER_W, D), jnp.float32),
            pltpu.SemaphoreType.DMA,
        ],
    )
    return f(table, idx)


def _sc_vals_body(table_hbm, idx_hbm, out_hbm, idx_v, rows_v, sem):
    wid = lax.axis_index("s") * 2 + lax.axis_index("c")
    base = wid * ROWS_PER_W * MAX_K
    pltpu.sync_copy(idx_hbm.at[pl.ds(base, ROWS_PER_W * MAX_K)], idx_v)
    for c in range(MAX_K):
        pltpu.async_copy(table_hbm.at[idx_v.at[pl.ds(c * 64, 64)]],
                         rows_v, sem).wait()
        pltpu.sync_copy(rows_v, out_hbm.at[pl.ds(base + c * 64, 64)])


def _sc_vals_gather(table, idx_flat):
    mesh = plsc.VectorSubcoreMesh(core_axis_name="c", subcore_axis_name="s")
    f = pl.kernel(
        _sc_vals_body,
        mesh=mesh,
        out_type=jax.ShapeDtypeStruct((S * MAX_K, D), jnp.float32),
        scratch_types=[
            pltpu.VMEM((ROWS_PER_W * MAX_K,), jnp.int32),
            pltpu.VMEM((64, D), jnp.float32),
            pltpu.SemaphoreType.DMA,
        ],
    )
    return f(table, idx_flat)


def _router_body(h_ref, scale_ref, bias_ref, wq_ref, bq_ref, keys_ref,
                 gated_ref, idx_ref, q_scr, rv_scr, ri_scr):
    j = pl.program_id(0)
    nj = pl.num_programs(0)
    t = pl.program_id(1)
    rows = pl.ds(t * T_TILE, T_TILE)

    @pl.when(j == 0)
    def _ln_q():
        h = h_ref[...]
        mu = jnp.mean(h, axis=-1, keepdims=True)
        var = jnp.mean((h - mu) ** 2, axis=-1, keepdims=True)
        hn = (h - mu) / jnp.sqrt(var + 1e-6) * scale_ref[...] + bias_ref[...]
        q_scr[rows, :] = jnp.dot(hn, wq_ref[...],
                                 preferred_element_type=jnp.float32) + bq_ref[...]
        rv_scr[rows, :] = jnp.full((T_TILE, MAX_K), NEG, jnp.float32)
        ri_scr[rows, :] = jnp.zeros((T_TILE, MAX_K), jnp.int32)

    s = jax.lax.dot_general(q_scr[rows, :], keys_ref[...],
                            (((1,), (1,)), ((), ())),
                            preferred_element_type=jnp.float32) * (1.0 / 32.0)

    iota = jax.lax.broadcasted_iota(jnp.int32, s.shape, 1)
    bv, bi = [], []
    for _ in range(MAX_K):
        m = jnp.max(s, axis=1, keepdims=True)
        am = jnp.min(jnp.where(s == m, iota, jnp.int32(1 << 30)),
                     axis=1, keepdims=True)
        bv.append(m)
        bi.append(am + j * KEY_TILE)
        s = jnp.where(iota == am, NEG, s)

    cv = jnp.concatenate([rv_scr[rows, :]] + bv, axis=1)
    ci = jnp.concatenate([ri_scr[rows, :]] + bi, axis=1)
    pos = jax.lax.broadcasted_iota(jnp.int32, cv.shape, 1)
    nv, ni = [], []
    for _ in range(MAX_K):
        m = jnp.max(cv, axis=1, keepdims=True)
        ap = jnp.min(jnp.where(cv == m, pos, jnp.int32(1 << 30)),
                     axis=1, keepdims=True)
        nv.append(m)
        ni.append(jnp.sum(jnp.where(pos == ap, ci, 0), axis=1, keepdims=True))
        cv = jnp.where(pos == ap, NEG, cv)
    rv_scr[rows, :] = jnp.concatenate(nv, axis=1)
    ri_scr[rows, :] = jnp.concatenate(ni, axis=1)

    @pl.when(j == nj - 1)
    def _gate():
        tv = rv_scr[rows, :]
        e = jnp.exp(tv - tv[:, 0:1])
        p = e / jnp.sum(e, axis=1, keepdims=True)
        cums = [p[:, 0:1]]
        for k in range(1, MAX_K - 1):
            cums.append(cums[-1] + p[:, k:k + 1])
        prev = jnp.concatenate([jnp.zeros_like(cums[0])] + cums, axis=1)
        lane = jax.lax.broadcasted_iota(jnp.int32, p.shape, 1)
        keep = (lane < MIN_K) | (prev < TAU)
        g = p * keep.astype(jnp.float32)
        g = g / (jnp.sum(g, axis=1, keepdims=True) + 1e-9)
        gated_ref[...] = g
        idx_ref[...] = ri_scr[rows, :]


def _router(h, lp):
    grid = (M // KEY_TILE, S // T_TILE)
    return pl.pallas_call(
        _router_body,
        grid=grid,
        in_specs=[
            pl.BlockSpec((T_TILE, D),
                         lambda j, t: (jnp.where(j == 0, t, S // T_TILE - 1), 0)),
            pl.BlockSpec((1, D), lambda j, t: (0, 0)),
            pl.BlockSpec((1, D), lambda j, t: (0, 0)),
            pl.BlockSpec((D, D), lambda j, t: (0, 0)),
            pl.BlockSpec((1, D), lambda j, t: (0, 0)),
            pl.BlockSpec((KEY_TILE, D), lambda j, t: (j, 0)),
        ],
        out_specs=[
            pl.BlockSpec((T_TILE, MAX_K),
                         lambda j, t: (jnp.where(j == M // KEY_TILE - 1, t, 0), 0)),
            pl.BlockSpec((T_TILE, MAX_K),
                         lambda j, t: (jnp.where(j == M // KEY_TILE - 1, t, 0), 0)),
        ],
        out_shape=[
            jax.ShapeDtypeStruct((S, MAX_K), jnp.float32),
            jax.ShapeDtypeStruct((S, MAX_K), jnp.int32),
        ],
        scratch_shapes=[
            pltpu.VMEM((S, D), jnp.float32),
            pltpu.VMEM((S, MAX_K), jnp.float32),
            pltpu.VMEM((S, MAX_K), jnp.int32),
        ],
        compiler_params=pltpu.CompilerParams(
            dimension_semantics=("arbitrary", "arbitrary")),
    )(h, lp['ln_scale'].reshape(1, D), lp['ln_bias'].reshape(1, D),
      lp['Wq'], lp['bq'].reshape(1, D), lp['keys'])


def _combine_body(vals_ref, g_ref, h_ref, wo_ref, bo_ref, out_ref):
    g = g_ref[...].astype(jnp.bfloat16).astype(jnp.float32)
    def vk(k):
        return vals_ref[:, k, :].astype(jnp.bfloat16).astype(jnp.float32)
    mem = vk(0) * g[:, 0:1]
    for k in range(1, MAX_K):
        mem = mem + vk(k) * g[:, k:k + 1]
    out_ref[...] = (jnp.dot(mem, wo_ref[...],
                            preferred_element_type=jnp.float32)
                    + bo_ref[...] + h_ref[...])


def _combine(vals, gated, h, lp):
    grid = (S // T_TILE,)
    return pl.pallas_call(
        _combine_body,
        grid=grid,
        in_specs=[
            pl.BlockSpec((T_TILE, MAX_K, D), lambda t: (t, 0, 0)),
            pl.BlockSpec((T_TILE, MAX_K), lambda t: (t, 0)),
            pl.BlockSpec((T_TILE, D), lambda t: (t, 0)),
            pl.BlockSpec((D, D), lambda t: (0, 0)),
            pl.BlockSpec((1, D), lambda t: (0, 0)),
        ],
        out_specs=pl.BlockSpec((T_TILE, D), lambda t: (t, 0)),
        out_shape=jax.ShapeDtypeStruct((S, D), jnp.float32),
        compiler_params=pltpu.CompilerParams(
            dimension_semantics=("parallel",)),
    )(vals, gated, h, lp['Wo'], lp['bo'].reshape(1, D))


def _final_body(h_ref, scale_ref, bias_ref, wout_ref, bout_ref, out_ref, hn_scr):
    v = pl.program_id(0)

    @pl.when(v == 0)
    def _ln():
        h = h_ref[...]
        mu = jnp.mean(h, axis=-1, keepdims=True)
        var = jnp.mean((h - mu) ** 2, axis=-1, keepdims=True)
        hn_scr[...] = ((h - mu) / jnp.sqrt(var + 1e-6) * scale_ref[...]
                       + bias_ref[...])

    out_ref[...] = (jnp.dot(hn_scr[...], wout_ref[...],
                            preferred_element_type=jnp.float32) + bout_ref[...])


def _final(h, params):
    vocab = params['Wout'].shape[1]
    grid = (vocab // V_TILE,)
    return pl.pallas_call(
        _final_body,
        grid=grid,
        in_specs=[
            pl.BlockSpec((S, D), lambda v: (0, 0)),
            pl.BlockSpec((1, D), lambda v: (0, 0)),
            pl.BlockSpec((1, D), lambda v: (0, 0)),
            pl.BlockSpec((D, V_TILE), lambda v: (0, v)),
            pl.BlockSpec((1, V_TILE), lambda v: (0, v)),
        ],
        out_specs=pl.BlockSpec((S, V_TILE), lambda v: (0, v)),
        out_shape=jax.ShapeDtypeStruct((S, vocab), jnp.float32),
        scratch_shapes=[pltpu.VMEM((S, D), jnp.float32)],
        compiler_params=pltpu.CompilerParams(
            dimension_semantics=("arbitrary",)),
    )(h, params['f_ln_scale'].reshape(1, D), params['f_ln_bias'].reshape(1, D),
      params['Wout'], params['bout'].reshape(1, vocab))


def _aux_body(idx_ref, g_ref, aux_ref, imp_scr):
    t = pl.program_id(0)
    nt = pl.num_programs(0)

    @pl.when(t == 0)
    def _init():
        imp_scr[...] = jnp.zeros_like(imp_scr)

    idx = idx_ref[...]
    g = g_ref[...]
    hi = jax.lax.shift_right_logical(idx, 7)
    lo = jax.lax.bitwise_and(idx, 127)
    acols, bcols = [], []
    for k in range(MAX_K):
        hi_k, lo_k, g_k = hi[:, k:k + 1], lo[:, k:k + 1], g[:, k:k + 1]
        i64 = jax.lax.broadcasted_iota(jnp.int32, (hi.shape[0], 64), 1)
        i128 = jax.lax.broadcasted_iota(jnp.int32, (hi.shape[0], 128), 1)
        acols.append((hi_k == i64).astype(jnp.float32) * g_k)
        bcols.append((lo_k == i128).astype(jnp.float32))
    a = jnp.concatenate(acols, axis=0)
    b = jnp.concatenate(bcols, axis=0)
    imp_scr[...] += jax.lax.dot_general(
        a, b, (((0,), (0,)), ((), ())),
        preferred_element_type=jnp.float32, precision=HI)

    @pl.when(t == nt - 1)
    def _fin():
        imp = imp_scr[...] / float(S)
        mean = jnp.mean(imp)
        aux_ref[...] = (jnp.sum((imp - mean) ** 2) * float(M)).reshape(1, 1)


def _aux(top_idx, gated):
    return pl.pallas_call(
        _aux_body,
        grid=(S // T_TILE,),
        in_specs=[
            pl.BlockSpec((T_TILE, MAX_K), lambda t: (t, 0)),
            pl.BlockSpec((T_TILE, MAX_K), lambda t: (t, 0)),
        ],
        out_specs=pl.BlockSpec((1, 1), lambda t: (0, 0)),
        out_shape=jax.ShapeDtypeStruct((1, 1), jnp.float32),
        scratch_shapes=[pltpu.VMEM((64, 128), jnp.float32)],
        compiler_params=pltpu.CompilerParams(
            dimension_semantics=("arbitrary",)),
    )(top_idx, gated)


def kernel(x, params):
    tokens = x.reshape(-1)
    h = _sc_embed_gather(params['embed'], tokens)
    aux_total = jnp.float32(0.0)
    for lp in params['layers']:
        gated, top_idx = _router(h, lp)
        vals = _sc_vals_gather(lp['values'], top_idx.reshape(-1))
        h = _combine(vals.reshape(S, MAX_K, D), gated, h, lp)
        aux_total = aux_total + _aux(top_idx, gated)[0, 0]
    logits = _final(h, params)
    return logits.reshape(1, S, -1), aux_total


# speedup vs baseline: 1.2817x; 1.0311x over previous
import functools

import jax
import jax.numpy as jnp
from jax import lax
from jax.experimental import pallas as pl
from jax.experimental.pallas import tpu as pltpu
from jax.experimental.pallas import tpu_sc as plsc

D = 1024
M = 8192
S = 2048
MAX_K = 8
MIN_K = 2
TAU = 0.9

T_TILE = 256
KEY_TILE = 1024
V_TILE = 1280
NEG = -3.0e38
HI = jax.lax.Precision.HIGHEST

NW = 32
ROWS_PER_W = S // NW


def _sc_embed_body(table_hbm, idx_hbm, out_hbm, idx_v, rows_v, sem):
    wid = lax.axis_index("s") * 2 + lax.axis_index("c")
    base = wid * ROWS_PER_W
    pltpu.sync_copy(idx_hbm.at[pl.ds(base, ROWS_PER_W)], idx_v)
    pltpu.async_copy(table_hbm.at[idx_v], rows_v, sem).wait()
    pltpu.sync_copy(rows_v, out_hbm.at[pl.ds(base, ROWS_PER_W)])


def _sc_embed_gather(table, idx):
    mesh = plsc.VectorSubcoreMesh(core_axis_name="c", subcore_axis_name="s")
    f = pl.kernel(
        _sc_embed_body,
        mesh=mesh,
        out_type=jax.ShapeDtypeStruct((S, D), jnp.float32),
        scratch_types=[
            pltpu.VMEM((ROWS_PER_W,), jnp.int32),
            pltpu.VMEM((ROWS_PER_W, D), jnp.float32),
            pltpu.SemaphoreType.DMA,
        ],
    )
    return f(table, idx)


def _sc_vals_body(table_hbm, idx_hbm, out_hbm, idx_v, rows_v, sem0, sem1):
    wid = lax.axis_index("s") * 2 + lax.axis_index("c")
    base = wid * ROWS_PER_W * MAX_K
    pltpu.sync_copy(idx_hbm.at[pl.ds(base, ROWS_PER_W * MAX_K)], idx_v)
    sems = [sem0, sem1]
    nch = (ROWS_PER_W * MAX_K) // 32
    prev = None
    for c in range(nch):
        cur = pltpu.async_copy(table_hbm.at[idx_v.at[pl.ds(c * 32, 32)]],
                               rows_v.at[c % 2], sems[c % 2])
        if prev is not None:
            pc, ph = prev
            ph.wait()
            pltpu.sync_copy(rows_v.at[pc % 2],
                            out_hbm.at[pl.ds(base + pc * 32, 32)])
        prev = (c, cur)
    pc, ph = prev
    ph.wait()
    pltpu.sync_copy(rows_v.at[pc % 2], out_hbm.at[pl.ds(base + pc * 32, 32)])


def _sc_vals_gather(table, idx_flat):
    mesh = plsc.VectorSubcoreMesh(core_axis_name="c", subcore_axis_name="s")
    f = pl.kernel(
        _sc_vals_body,
        mesh=mesh,
        out_type=jax.ShapeDtypeStruct((S * MAX_K, D), jnp.float32),
        scratch_types=[
            pltpu.VMEM((ROWS_PER_W * MAX_K,), jnp.int32),
            pltpu.VMEM((2, 32, D), jnp.float32),
            pltpu.SemaphoreType.DMA,
            pltpu.SemaphoreType.DMA,
        ],
    )
    return f(table, idx_flat)


def _router_body(h_ref, scale_ref, bias_ref, wq_ref, bq_ref, keys_ref,
                 gated_ref, idx_ref, q_scr, rv_scr, ri_scr):
    j = pl.program_id(0)
    nj = pl.num_programs(0)
    t = pl.program_id(1)
    rows = pl.ds(t * T_TILE, T_TILE)

    @pl.when(j == 0)
    def _ln_q():
        h = h_ref[...]
        mu = jnp.mean(h, axis=-1, keepdims=True)
        var = jnp.mean((h - mu) ** 2, axis=-1, keepdims=True)
        hn = (h - mu) / jnp.sqrt(var + 1e-6) * scale_ref[...] + bias_ref[...]
        q_scr[rows, :] = jnp.dot(hn, wq_ref[...],
                                 preferred_element_type=jnp.float32) + bq_ref[...]
        rv_scr[rows, :] = jnp.full((T_TILE, MAX_K), NEG, jnp.float32)
        ri_scr[rows, :] = jnp.zeros((T_TILE, MAX_K), jnp.int32)

    s = jax.lax.dot_general(q_scr[rows, :], keys_ref[...],
                            (((1,), (1,)), ((), ())),
                            preferred_element_type=jnp.float32) * (1.0 / 32.0)

    iota = jax.lax.broadcasted_iota(jnp.int32, s.shape, 1)
    bv, bi = [], []
    for _ in range(MAX_K):
        m = jnp.max(s, axis=1, keepdims=True)
        am = jnp.min(jnp.where(s == m, iota, jnp.int32(1 << 30)),
                     axis=1, keepdims=True)
        bv.append(m)
        bi.append(am + j * KEY_TILE)
        s = jnp.where(iota == am, NEG, s)

    cv = jnp.concatenate([rv_scr[rows, :]] + bv, axis=1)
    ci = jnp.concatenate([ri_scr[rows, :]] + bi, axis=1)
    pos = jax.lax.broadcasted_iota(jnp.int32, cv.shape, 1)
    nv, ni = [], []
    for _ in range(MAX_K):
        m = jnp.max(cv, axis=1, keepdims=True)
        ap = jnp.min(jnp.where(cv == m, pos, jnp.int32(1 << 30)),
                     axis=1, keepdims=True)
        nv.append(m)
        ni.append(jnp.sum(jnp.where(pos == ap, ci, 0), axis=1, keepdims=True))
        cv = jnp.where(pos == ap, NEG, cv)
    rv_scr[rows, :] = jnp.concatenate(nv, axis=1)
    ri_scr[rows, :] = jnp.concatenate(ni, axis=1)

    @pl.when(j == nj - 1)
    def _gate():
        tv = rv_scr[rows, :]
        e = jnp.exp(tv - tv[:, 0:1])
        p = e / jnp.sum(e, axis=1, keepdims=True)
        cums = [p[:, 0:1]]
        for k in range(1, MAX_K - 1):
            cums.append(cums[-1] + p[:, k:k + 1])
        prev = jnp.concatenate([jnp.zeros_like(cums[0])] + cums, axis=1)
        lane = jax.lax.broadcasted_iota(jnp.int32, p.shape, 1)
        keep = (lane < MIN_K) | (prev < TAU)
        g = p * keep.astype(jnp.float32)
        g = g / (jnp.sum(g, axis=1, keepdims=True) + 1e-9)
        gated_ref[...] = g
        idx_ref[...] = ri_scr[rows, :]


def _router(h, lp):
    grid = (M // KEY_TILE, S // T_TILE)
    return pl.pallas_call(
        _router_body,
        grid=grid,
        in_specs=[
            pl.BlockSpec((T_TILE, D),
                         lambda j, t: (jnp.where(j == 0, t, S // T_TILE - 1), 0)),
            pl.BlockSpec((1, D), lambda j, t: (0, 0)),
            pl.BlockSpec((1, D), lambda j, t: (0, 0)),
            pl.BlockSpec((D, D), lambda j, t: (0, 0)),
            pl.BlockSpec((1, D), lambda j, t: (0, 0)),
            pl.BlockSpec((KEY_TILE, D), lambda j, t: (j, 0)),
        ],
        out_specs=[
            pl.BlockSpec((T_TILE, MAX_K),
                         lambda j, t: (jnp.where(j == M // KEY_TILE - 1, t, 0), 0)),
            pl.BlockSpec((T_TILE, MAX_K),
                         lambda j, t: (jnp.where(j == M // KEY_TILE - 1, t, 0), 0)),
        ],
        out_shape=[
            jax.ShapeDtypeStruct((S, MAX_K), jnp.float32),
            jax.ShapeDtypeStruct((S, MAX_K), jnp.int32),
        ],
        scratch_shapes=[
            pltpu.VMEM((S, D), jnp.float32),
            pltpu.VMEM((S, MAX_K), jnp.float32),
            pltpu.VMEM((S, MAX_K), jnp.int32),
        ],
        compiler_params=pltpu.CompilerParams(
            dimension_semantics=("arbitrary", "arbitrary")),
    )(h, lp['ln_scale'].reshape(1, D), lp['ln_bias'].reshape(1, D),
      lp['Wq'], lp['bq'].reshape(1, D), lp['keys'])


def _combine_body(vals_ref, g_ref, h_ref, wo_ref, bo_ref, out_ref):
    g = g_ref[...].astype(jnp.bfloat16).astype(jnp.float32)
    def vk(k):
        return vals_ref[:, k, :].astype(jnp.bfloat16).astype(jnp.float32)
    mem = vk(0) * g[:, 0:1]
    for k in range(1, MAX_K):
        mem = mem + vk(k) * g[:, k:k + 1]
    out_ref[...] = (jnp.dot(mem, wo_ref[...],
                            preferred_element_type=jnp.float32)
                    + bo_ref[...] + h_ref[...])


def _combine(vals, gated, h, lp):
    grid = (S // T_TILE,)
    return pl.pallas_call(
        _combine_body,
        grid=grid,
        in_specs=[
            pl.BlockSpec((T_TILE, MAX_K, D), lambda t: (t, 0, 0)),
            pl.BlockSpec((T_TILE, MAX_K), lambda t: (t, 0)),
            pl.BlockSpec((T_TILE, D), lambda t: (t, 0)),
            pl.BlockSpec((D, D), lambda t: (0, 0)),
            pl.BlockSpec((1, D), lambda t: (0, 0)),
        ],
        out_specs=pl.BlockSpec((T_TILE, D), lambda t: (t, 0)),
        out_shape=jax.ShapeDtypeStruct((S, D), jnp.float32),
        compiler_params=pltpu.CompilerParams(
            dimension_semantics=("parallel",)),
    )(vals, gated, h, lp['Wo'], lp['bo'].reshape(1, D))


def _final_body(h_ref, scale_ref, bias_ref, wout_ref, bout_ref, out_ref, hn_scr):
    v = pl.program_id(0)

    @pl.when(v == 0)
    def _ln():
        h = h_ref[...]
        mu = jnp.mean(h, axis=-1, keepdims=True)
        var = jnp.mean((h - mu) ** 2, axis=-1, keepdims=True)
        hn_scr[...] = ((h - mu) / jnp.sqrt(var + 1e-6) * scale_ref[...]
                       + bias_ref[...])

    out_ref[...] = (jnp.dot(hn_scr[...], wout_ref[...],
                            preferred_element_type=jnp.float32) + bout_ref[...])


def _final(h, params):
    vocab = params['Wout'].shape[1]
    grid = (vocab // V_TILE,)
    return pl.pallas_call(
        _final_body,
        grid=grid,
        in_specs=[
            pl.BlockSpec((S, D), lambda v: (0, 0)),
            pl.BlockSpec((1, D), lambda v: (0, 0)),
            pl.BlockSpec((1, D), lambda v: (0, 0)),
            pl.BlockSpec((D, V_TILE), lambda v: (0, v)),
            pl.BlockSpec((1, V_TILE), lambda v: (0, v)),
        ],
        out_specs=pl.BlockSpec((S, V_TILE), lambda v: (0, v)),
        out_shape=jax.ShapeDtypeStruct((S, vocab), jnp.float32),
        scratch_shapes=[pltpu.VMEM((S, D), jnp.float32)],
        compiler_params=pltpu.CompilerParams(
            dimension_semantics=("arbitrary",)),
    )(h, params['f_ln_scale'].reshape(1, D), params['f_ln_bias'].reshape(1, D),
      params['Wout'], params['bout'].reshape(1, vocab))


def _aux_body(idx_ref, g_ref, aux_ref, imp_scr):
    t = pl.program_id(0)
    nt = pl.num_programs(0)

    @pl.when(t == 0)
    def _init():
        imp_scr[...] = jnp.zeros_like(imp_scr)

    idx = idx_ref[...]
    g = g_ref[...]
    hi = jax.lax.shift_right_logical(idx, 7)
    lo = jax.lax.bitwise_and(idx, 127)
    acols, bcols = [], []
    for k in range(MAX_K):
        hi_k, lo_k, g_k = hi[:, k:k + 1], lo[:, k:k + 1], g[:, k:k + 1]
        i64 = jax.lax.broadcasted_iota(jnp.int32, (hi.shape[0], 64), 1)
        i128 = jax.lax.broadcasted_iota(jnp.int32, (hi.shape[0], 128), 1)
        acols.append((hi_k == i64).astype(jnp.float32) * g_k)
        bcols.append((lo_k == i128).astype(jnp.float32))
    a = jnp.concatenate(acols, axis=0)
    b = jnp.concatenate(bcols, axis=0)
    imp_scr[...] += jax.lax.dot_general(
        a, b, (((0,), (0,)), ((), ())),
        preferred_element_type=jnp.float32, precision=HI)

    @pl.when(t == nt - 1)
    def _fin():
        imp = imp_scr[...] / float(S)
        mean = jnp.mean(imp)
        aux_ref[...] = (jnp.sum((imp - mean) ** 2) * float(M)).reshape(1, 1)


def _aux(top_idx, gated):
    return pl.pallas_call(
        _aux_body,
        grid=(S // T_TILE,),
        in_specs=[
            pl.BlockSpec((T_TILE, MAX_K), lambda t: (t, 0)),
            pl.BlockSpec((T_TILE, MAX_K), lambda t: (t, 0)),
        ],
        out_specs=pl.BlockSpec((1, 1), lambda t: (0, 0)),
        out_shape=jax.ShapeDtypeStruct((1, 1), jnp.float32),
        scratch_shapes=[pltpu.VMEM((64, 128), jnp.float32)],
        compiler_params=pltpu.CompilerParams(
            dimension_semantics=("arbitrary",)),
    )(top_idx, gated)


def kernel(x, params):
    tokens = x.reshape(-1)
    h = _sc_embed_gather(params['embed'], tokens)
    aux_total = jnp.float32(0.0)
    for lp in params['layers']:
        gated, top_idx = _router(h, lp)
        vals = _sc_vals_gather(lp['values'], top_idx.reshape(-1))
        h = _combine(vals.reshape(S, MAX_K, D), gated, h, lp)
        aux_total = aux_total + _aux(top_idx, gated)[0, 0]
    logits = _final(h, params)
    return logits.reshape(1, S, -1), aux_total
